# Initial kernel scaffold; baseline (speedup 1.0000x reference)
import jax
import jax.numpy as jnp
from jax.experimental import pallas as pl


def kernel(topK_label_vals, label_shortlist_vals, y_vals, topK_label_inds, label_shortlist_inds, y_inds):
    raise NotImplementedError("write your pallas kernel here")


# speedup vs baseline: 6.8220x; 6.8220x over previous
import functools

import jax
import jax.numpy as jnp
from jax import lax
from jax.experimental import pallas as pl
from jax.experimental.pallas import tpu as pltpu
from jax.experimental.pallas import tpu_sc as plsc

_B, _K, _S, _LY = 4096, 100, 2000, 20
_KP = 112
_LYP = 32
_POSW = 32
_NC, _NS = 2, 16
_NW = _NC * _NS
_CH = 16
_RPW = _B // _NW
_NCH = _RPW // _CH
_LAMBDA = 0.05


def _sc_body(tki_h, si_h, sv_h, yi_h, yv_h,
             tkt_h, posv_h, post_h,
             tki, si, sv, yi, yv, tkt, posv, post):
    wid = lax.axis_index("s") * _NC + lax.axis_index("c")

    def chunk_body(ci, carry):
        r0 = wid * _RPW + ci * _CH
        pltpu.sync_copy(tki_h.at[pl.ds(r0, _CH)], tki)
        pltpu.sync_copy(si_h.at[pl.ds(r0, _CH)], si)
        pltpu.sync_copy(sv_h.at[pl.ds(r0, _CH)], sv)
        pltpu.sync_copy(yi_h.at[pl.ds(r0, _CH)], yi)
        pltpu.sync_copy(yv_h.at[pl.ds(r0, _CH)], yv)

        def row_body(r, carry2):
            z = jnp.zeros((16,), jnp.float32)
            posv[r, pl.ds(0, 16)] = z
            posv[r, pl.ds(16, 16)] = z
            post[r, pl.ds(0, 16)] = z
            post[r, pl.ds(16, 16)] = z

            yia = yi[r, pl.ds(0, 16)]
            yib = yi[r, pl.ds(16, 16)]
            yva = yv[r, pl.ds(0, 16)]
            yvb = yv[r, pl.ds(16, 16)]
            yis = [yia[j] for j in range(16)] + [yib[j] for j in range(_LY - 16)]
            yvs = [yva[j] for j in range(16)] + [yvb[j] for j in range(_LY - 16)]

            for v in range(_KP // 16):
                inds = tki[r, pl.ds(v * 16, 16)]
                t = z
                for j in range(_LY):
                    t = jnp.where(inds == yis[j], yvs[j], t)
                tkt[r, pl.ds(v * 16, 16)] = t

            one = jnp.ones((16,), jnp.float32)

            def cand(i, off):
                inds = si[r, pl.ds(i * 16, 16)]
                m = inds == yis[0]
                for j in range(1, _LY):
                    m = jnp.logical_or(m, inds == yis[j])
                rank = off + plsc.cumsum(jnp.where(m, 1, 0).astype(jnp.int32))
                wr = jnp.logical_and(m, rank <= _LY)
                vals = sv[r, pl.ds(i * 16, 16)]
                plsc.store_scatter(posv.at[r], [rank - 1], vals, mask=wr)
                plsc.store_scatter(post.at[r], [rank - 1], one, mask=wr)
                return off + plsc.all_reduce_population_count(m)

            lax.fori_loop(0, _S // 16, cand, jnp.zeros((16,), jnp.int32))
            return carry2

        lax.fori_loop(0, _CH, row_body, 0)
        pltpu.sync_copy(tkt, tkt_h.at[pl.ds(r0, _CH)])
        pltpu.sync_copy(posv, posv_h.at[pl.ds(r0, _CH)])
        pltpu.sync_copy(post, post_h.at[pl.ds(r0, _CH)])
        return carry

    lax.fori_loop(0, _NCH, chunk_body, 0)


@functools.cache
def _sc_match():
    return pl.kernel(
        _sc_body,
        out_type=(
            jax.ShapeDtypeStruct((_B, _KP), jnp.float32),
            jax.ShapeDtypeStruct((_B, _POSW), jnp.float32),
            jax.ShapeDtypeStruct((_B, _POSW), jnp.float32),
        ),
        mesh=plsc.VectorSubcoreMesh(core_axis_name="c", subcore_axis_name="s",
                                    num_cores=_NC, num_subcores=_NS),
        compiler_params=pltpu.CompilerParams(needs_layout_passes=False),
        scratch_types=[
            pltpu.VMEM((_CH, _KP), jnp.int32),
            pltpu.VMEM((_CH, _S), jnp.int32),
            pltpu.VMEM((_CH, _S), jnp.float32),
            pltpu.VMEM((_CH, _LYP), jnp.int32),
            pltpu.VMEM((_CH, _LYP), jnp.float32),
            pltpu.VMEM((_CH, _KP), jnp.float32),
            pltpu.VMEM((_CH, _POSW), jnp.float32),
            pltpu.VMEM((_CH, _POSW), jnp.float32),
        ],
    )


def _tc_body(p_ref, t_ref, v_ref, tt_ref, o_ref):
    p = p_ref[...]
    t = t_ref[...]
    lp = jnp.maximum(jnp.log(p), -100.0)
    l1p = jnp.maximum(jnp.log(1.0 - p), -100.0)
    term1 = -jnp.sum(t * lp + (1.0 - t) * l1p)
    v = v_ref[...]
    tt = tt_ref[...]
    lv = jnp.maximum(jnp.log(v), -100.0)
    l1v = jnp.maximum(jnp.log(1.0 - v), -100.0)
    term2 = -jnp.sum(tt * lv + (1.0 - tt) * l1v)
    total = term1 / (_B * _K) + _LAMBDA * term2 / (_B * _LY)
    o_ref[...] = total.reshape(1, 1)


_tc_bce = pl.pallas_call(
    _tc_body,
    out_shape=jax.ShapeDtypeStruct((1, 1), jnp.float32),
)


def kernel(topK_label_vals, label_shortlist_vals, y_vals,
           topK_label_inds, label_shortlist_inds, y_inds):
    tki = jnp.pad(topK_label_inds, ((0, 0), (0, _KP - _K)), constant_values=-1)
    yi = jnp.pad(y_inds, ((0, 0), (0, _LYP - _LY)), constant_values=-1)
    yv = jnp.pad(y_vals, ((0, 0), (0, _LYP - _LY)))
    tkt, posv, post = _sc_match()(
        tki, label_shortlist_inds, label_shortlist_vals, yi, yv)
    loss = _tc_bce(topK_label_vals, tkt[:, :_K], posv, post)
    return loss[0, 0]
